# baseline (speedup 1.0000x reference)
import functools

import jax
import jax.numpy as jnp
from jax import lax
from jax.experimental import pallas as pl
from jax.experimental.pallas import tpu as pltpu
from jax.experimental.pallas import tpu_sc as plsc

NWORDS = 1000000
NTAGS = 64
NUM_WORDS = 16384

NC = 2
NS = 16
NW = NC * NS
B_PER_W = NUM_WORDS // NW
LANES = 16

CPAD = 1 << 20
SHARD = CPAD // NS
ZCHUNK = 8192
IDX_MINOR = 128
NIDX = B_PER_W // IDX_MINOR

BK = 16384
KTAIL = 999424
SPB = 48
CW = 256
K0 = KTAIL - NW * CW * SPB
NSTEP_MAIN = K0 // BK
NGRP = CW // LANES
NBUF = 4


def _sc_counts(words):
    mesh = plsc.VectorSubcoreMesh(core_axis_name="c", subcore_axis_name="s")

    @functools.partial(
        pl.kernel,
        mesh=mesh,
        out_type=jax.ShapeDtypeStruct((NC, CPAD), jnp.float32),
        scratch_types=[
            pltpu.VMEM((ZCHUNK,), jnp.float32),
            pltpu.VMEM((NIDX, IDX_MINOR), jnp.int32),
            pltpu.VMEM((IDX_MINOR,), jnp.float32),
            pltpu.VMEM_SHARED((CPAD,), jnp.float32),
        ],
    )
    def k(words_hbm, out_hbm, zeros_v, idx_v, ones_v, counts_sh):
        cid = lax.axis_index("c")
        sid = lax.axis_index("s")
        wid = sid * NC + cid
        base = wid * B_PER_W

        for j in range(NIDX):
            pltpu.sync_copy(
                words_hbm.at[pl.ds(base + j * IDX_MINOR, IDX_MINOR)],
                idx_v.at[j],
            )
        for t in range(ZCHUNK // LANES):
            zeros_v[pl.ds(t * LANES, LANES)] = jnp.zeros((LANES,), jnp.float32)
        for t in range(IDX_MINOR // LANES):
            ones_v[pl.ds(t * LANES, LANES)] = jnp.ones((LANES,), jnp.float32)

        for t in range(SHARD // ZCHUNK):
            pltpu.sync_copy(
                zeros_v, counts_sh.at[pl.ds(sid * SHARD + t * ZCHUNK, ZCHUNK)]
            )
        plsc.subcore_barrier()

        for j in range(NIDX):
            pltpu.sync_copy(ones_v, counts_sh.at[idx_v.at[j]], add=True)
        plsc.subcore_barrier()

        pltpu.sync_copy(
            counts_sh.at[pl.ds(sid * SHARD, SHARD)],
            out_hbm.at[cid, pl.ds(sid * SHARD, SHARD)],
        )

    return k(words)


def _sc_matvec(xt, counts):
    mesh = plsc.VectorSubcoreMesh(core_axis_name="c", subcore_axis_name="s")

    @functools.partial(
        pl.kernel,
        mesh=mesh,
        out_type=jax.ShapeDtypeStruct((NW, NTAGS, LANES), jnp.float32),
        scratch_types=(
            [pltpu.VMEM((NTAGS, CW), jnp.float32)] * NBUF
            + [
                pltpu.VMEM((NC, CW * SPB), jnp.float32),
                pltpu.VMEM((CW * SPB,), jnp.float32),
                pltpu.VMEM((NTAGS, LANES), jnp.float32),
            ]
            + [pltpu.SemaphoreType.DMA] * NBUF
        ),
    )
    def k(xt_hbm, cnt_hbm, out_hbm, *scr):
        xbs = scr[0:NBUF]
        cbig_v = scr[NBUF]
        csum_v = scr[NBUF + 1]
        acc_v = scr[NBUF + 2]
        sems = scr[NBUF + 3:]
        wid = lax.axis_index("s") * NC + lax.axis_index("c")
        wbase = K0 + wid * (CW * SPB)

        for t in range(NTAGS):
            acc_v[t, pl.ds(0, LANES)] = jnp.zeros((LANES,), jnp.float32)

        def fire(c, par):
            col = pl.multiple_of(wbase + c * CW, CW)
            pltpu.async_copy(xt_hbm.at[:, pl.ds(col, CW)], xbs[par], sems[par])

        def drain(par):
            pltpu.make_async_copy(
                xt_hbm.at[:, pl.ds(0, CW)], xbs[par], sems[par]
            ).wait()

        pltpu.sync_copy(cnt_hbm.at[:, pl.ds(wbase, CW * SPB)], cbig_v)

        def presum(i, carry):
            csum_v[pl.ds(i * LANES, LANES)] = (
                cbig_v[0, pl.ds(i * LANES, LANES)]
                + cbig_v[1, pl.ds(i * LANES, LANES)]
            )
            return carry

        lax.fori_loop(0, CW * SPB // LANES, presum, 0, unroll=8)

        def compute(c, par):
            xb = xbs[par]
            cs = tuple(
                csum_v[pl.ds(c * CW + g * LANES, LANES)] for g in range(NGRP)
            )

            def tag(t, carry):
                a = acc_v[t, pl.ds(0, LANES)]
                for g in range(NGRP):
                    a = a + xb[t, pl.ds(g * LANES, LANES)] * cs[g]
                acc_v[t, pl.ds(0, LANES)] = a
                return carry

            lax.fori_loop(0, NTAGS, tag, 0, unroll=8)

        for par in range(NBUF):
            fire(par, par)

        def ring(cc, carry):
            c = cc * NBUF
            for par in range(NBUF):
                drain(par)
                compute(c + par, par)
                fire(c + par + NBUF, par)
            return carry

        lax.fori_loop(0, SPB // NBUF - 1, ring, 0)
        c_last = SPB - NBUF
        for par in range(NBUF):
            drain(par)
            compute(c_last + par, par)

        pltpu.sync_copy(acc_v, out_hbm.at[wid])

    return k(xt, counts)


def _tc_matvec_kernel(xt_ref, cnt_ref, b_ref, o_ref, acc_ref):
    step = pl.program_id(0)

    @pl.when(step == 0)
    def _init():
        acc_ref[...] = jnp.zeros_like(acc_ref)

    cnt = (cnt_ref[0, :] + cnt_ref[1, :])[None, :]
    prod = jnp.where(cnt != 0.0, xt_ref[...] * cnt, 0.0)
    acc_ref[...] += jnp.sum(
        prod.reshape(NTAGS, BK // 128, 128), axis=1
    )

    @pl.when(step == NSTEP_MAIN)
    def _done():
        o_ref[...] = jnp.sum(acc_ref[...], axis=1)[None, :] + b_ref[...]


def _tc_combine_kernel(t_ref, s_ref, o_ref):
    o_ref[...] = t_ref[...] + jnp.sum(s_ref[...], axis=(0, 2))[None, :]


def kernel(words, emb_weight, bias):
    counts = _sc_counts(words.astype(jnp.int32))
    xt = emb_weight.T
    sc_part = _sc_matvec(xt, counts)
    tc_part = pl.pallas_call(
        _tc_matvec_kernel,
        grid=(NSTEP_MAIN + 1,),
        in_specs=[
            pl.BlockSpec(
                (NTAGS, BK),
                lambda k: (0, jnp.where(k == NSTEP_MAIN, KTAIL // BK, k)),
            ),
            pl.BlockSpec(
                (NC, BK),
                lambda k: (0, jnp.where(k == NSTEP_MAIN, KTAIL // BK, k)),
            ),
            pl.BlockSpec((1, NTAGS), lambda k: (0, 0)),
        ],
        out_specs=pl.BlockSpec((1, NTAGS), lambda k: (0, 0)),
        scratch_shapes=[pltpu.VMEM((NTAGS, 128), jnp.float32)],
        out_shape=jax.ShapeDtypeStruct((1, NTAGS), jnp.float32),
    )(xt, counts, bias.reshape(1, NTAGS))
    out = pl.pallas_call(
        _tc_combine_kernel,
        out_shape=jax.ShapeDtypeStruct((1, NTAGS), jnp.float32),
    )(tc_part, sc_part)
    return out


# speedup vs baseline: 1.0307x; 1.0307x over previous
import functools

import jax
import jax.numpy as jnp
from jax import lax
from jax.experimental import pallas as pl
from jax.experimental.pallas import tpu as pltpu
from jax.experimental.pallas import tpu_sc as plsc

NWORDS = 1000000
NTAGS = 64
NUM_WORDS = 16384

NC = 2
NS = 16
NW = NC * NS
B_PER_W = NUM_WORDS // NW
LANES = 16

CPAD = 1 << 20
SHARD = CPAD // NS
ZCHUNK = 8192
IDX_MINOR = 128
NIDX = B_PER_W // IDX_MINOR

BK = 16384
KTAIL = 999424
SPB = 48
CW = 256
K0 = KTAIL - NW * CW * SPB
NSTEP_MAIN = K0 // BK
NGRP = CW // LANES
NBUF = 4


def _sc_counts(words):
    mesh = plsc.VectorSubcoreMesh(core_axis_name="c", subcore_axis_name="s")

    @functools.partial(
        pl.kernel,
        mesh=mesh,
        out_type=jax.ShapeDtypeStruct((NC, CPAD), jnp.float32),
        scratch_types=[
            pltpu.VMEM((ZCHUNK,), jnp.float32),
            pltpu.VMEM((NIDX, IDX_MINOR), jnp.int32),
            pltpu.VMEM((IDX_MINOR,), jnp.float32),
            pltpu.VMEM_SHARED((CPAD,), jnp.float32),
        ],
    )
    def k(words_hbm, out_hbm, zeros_v, idx_v, ones_v, counts_sh):
        cid = lax.axis_index("c")
        sid = lax.axis_index("s")
        wid = sid * NC + cid
        base = wid * B_PER_W

        for j in range(NIDX):
            pltpu.sync_copy(
                words_hbm.at[pl.ds(base + j * IDX_MINOR, IDX_MINOR)],
                idx_v.at[j],
            )
        for t in range(ZCHUNK // LANES):
            zeros_v[pl.ds(t * LANES, LANES)] = jnp.zeros((LANES,), jnp.float32)
        for t in range(IDX_MINOR // LANES):
            ones_v[pl.ds(t * LANES, LANES)] = jnp.ones((LANES,), jnp.float32)

        for t in range(SHARD // ZCHUNK):
            pltpu.sync_copy(
                zeros_v, counts_sh.at[pl.ds(sid * SHARD + t * ZCHUNK, ZCHUNK)]
            )
        plsc.subcore_barrier()

        for j in range(NIDX):
            pltpu.sync_copy(ones_v, counts_sh.at[idx_v.at[j]], add=True)
        plsc.subcore_barrier()

        pltpu.sync_copy(
            counts_sh.at[pl.ds(sid * SHARD, SHARD)],
            out_hbm.at[cid, pl.ds(sid * SHARD, SHARD)],
        )

    return k(words)


def _sc_matvec(xt, counts):
    mesh = plsc.VectorSubcoreMesh(core_axis_name="c", subcore_axis_name="s")

    @functools.partial(
        pl.kernel,
        mesh=mesh,
        out_type=jax.ShapeDtypeStruct((NW, NTAGS, LANES), jnp.float32),
        scratch_types=(
            [pltpu.VMEM((NTAGS, CW), jnp.float32)] * NBUF
            + [pltpu.VMEM((NC, CW), jnp.float32)] * NBUF
            + [pltpu.VMEM((NTAGS, LANES), jnp.float32)]
            + [pltpu.SemaphoreType.DMA] * NBUF
        ),
    )
    def k(xt_hbm, cnt_hbm, out_hbm, *scr):
        xbs = scr[0:NBUF]
        cbs = scr[NBUF:2 * NBUF]
        acc_v = scr[2 * NBUF]
        sems = scr[2 * NBUF + 1:]
        wid = lax.axis_index("s") * NC + lax.axis_index("c")
        wbase = K0 + wid * (CW * SPB)

        for t in range(NTAGS):
            acc_v[t, pl.ds(0, LANES)] = jnp.zeros((LANES,), jnp.float32)

        def fire(c, par):
            col = pl.multiple_of(wbase + c * CW, CW)
            pltpu.async_copy(xt_hbm.at[:, pl.ds(col, CW)], xbs[par], sems[par])
            pltpu.async_copy(cnt_hbm.at[:, pl.ds(col, CW)], cbs[par], sems[par])

        def drain(par):
            pltpu.make_async_copy(
                xt_hbm.at[:, pl.ds(0, CW)], xbs[par], sems[par]
            ).wait()
            pltpu.make_async_copy(
                cnt_hbm.at[:, pl.ds(0, CW)], cbs[par], sems[par]
            ).wait()

        def compute(c, par):
            del c
            xb, cb = xbs[par], cbs[par]
            cs = tuple(
                cb[0, pl.ds(g * LANES, LANES)] + cb[1, pl.ds(g * LANES, LANES)]
                for g in range(NGRP)
            )

            def tag(t, carry):
                a = acc_v[t, pl.ds(0, LANES)]
                for g in range(NGRP):
                    a = a + xb[t, pl.ds(g * LANES, LANES)] * cs[g]
                acc_v[t, pl.ds(0, LANES)] = a
                return carry

            lax.fori_loop(0, NTAGS, tag, 0, unroll=8)

        for par in range(NBUF):
            fire(par, par)

        def ring(cc, carry):
            c = cc * NBUF
            for par in range(NBUF):
                drain(par)
                compute(c + par, par)
                fire(c + par + NBUF, par)
            return carry

        lax.fori_loop(0, SPB // NBUF - 1, ring, 0)
        c_last = SPB - NBUF
        for par in range(NBUF):
            drain(par)
            compute(c_last + par, par)

        pltpu.sync_copy(acc_v, out_hbm.at[wid])

    return k(xt, counts)


def _tc_matvec_kernel(xt_ref, cnt_ref, b_ref, o_ref, acc_ref):
    step = pl.program_id(0)

    @pl.when(step == 0)
    def _init():
        acc_ref[...] = jnp.zeros_like(acc_ref)

    cnt = (cnt_ref[0, :] + cnt_ref[1, :])[None, :]
    prod = jnp.where(cnt != 0.0, xt_ref[...] * cnt, 0.0)
    acc_ref[...] += jnp.sum(
        prod.reshape(NTAGS, BK // 128, 128), axis=1
    )

    @pl.when(step == NSTEP_MAIN)
    def _done():
        o_ref[...] = jnp.sum(acc_ref[...], axis=1)[None, :] + b_ref[...]


def _tc_combine_kernel(t_ref, s_ref, o_ref):
    o_ref[...] = t_ref[...] + jnp.sum(s_ref[...], axis=(0, 2))[None, :]


def kernel(words, emb_weight, bias):
    counts = _sc_counts(words.astype(jnp.int32))
    xt = emb_weight.T
    sc_part = _sc_matvec(xt, counts)
    tc_part = pl.pallas_call(
        _tc_matvec_kernel,
        grid=(NSTEP_MAIN + 1,),
        in_specs=[
            pl.BlockSpec(
                (NTAGS, BK),
                lambda k: (0, jnp.where(k == NSTEP_MAIN, KTAIL // BK, k)),
            ),
            pl.BlockSpec(
                (NC, BK),
                lambda k: (0, jnp.where(k == NSTEP_MAIN, KTAIL // BK, k)),
            ),
            pl.BlockSpec((1, NTAGS), lambda k: (0, 0)),
        ],
        out_specs=pl.BlockSpec((1, NTAGS), lambda k: (0, 0)),
        scratch_shapes=[pltpu.VMEM((NTAGS, 128), jnp.float32)],
        out_shape=jax.ShapeDtypeStruct((1, NTAGS), jnp.float32),
    )(xt, counts, bias.reshape(1, NTAGS))
    out = pl.pallas_call(
        _tc_combine_kernel,
        out_shape=jax.ShapeDtypeStruct((1, NTAGS), jnp.float32),
    )(tc_part, sc_part)
    return out
